# baseline (speedup 1.0000x reference)
import functools

import jax
import jax.numpy as jnp
from jax import lax
from jax.experimental import pallas as pl
from jax.experimental.pallas import tpu as pltpu
from jax.experimental.pallas import tpu_sc as plsc

NC = 2
NS = 16
NW = NC * NS
CHUNK = 184
NBUF = 5


@functools.partial(jax.jit, static_argnames=("chunks_per_w",))
def _sc_gather(idx, table, *, chunks_per_w):
    n = idx.shape[0]
    d = table.shape[1]
    b_per_w = chunks_per_w * CHUNK
    assert chunks_per_w >= NBUF + 1
    assert n % 8 == 0 and b_per_w <= n
    stride = -(-(n - b_per_w) // (8 * (NW - 1))) * 8
    mesh = plsc.VectorSubcoreMesh(core_axis_name="c", subcore_axis_name="s")

    @functools.partial(
        pl.kernel,
        mesh=mesh,
        out_type=jax.ShapeDtypeStruct((n, d), jnp.float32),
        scratch_types=[
            pltpu.VMEM((b_per_w,), jnp.int32),
            pltpu.VMEM((NBUF, CHUNK, d), jnp.float32),
            pltpu.VMEM_SHARED(table.shape, jnp.float32),
            pltpu.SemaphoreType.DMA,
            pltpu.SemaphoreType.DMA,
        ],
    )
    def k(idx_hbm, table_hbm, out_hbm, idx_v, rows_v, table_sp, gsem, osem):
        sid = lax.axis_index("s")
        wid = sid * NC + lax.axis_index("c")
        base = pl.multiple_of(jnp.minimum(wid * stride, n - b_per_w), 8)

        @pl.when(sid == 0)
        def _stage():
            pltpu.sync_copy(table_hbm, table_sp)

        pltpu.sync_copy(idx_hbm.at[pl.ds(base, b_per_w)], idx_v)
        plsc.subcore_barrier()

        def gather(c, slot):
            pltpu.async_copy(
                table_sp.at[idx_v.at[pl.ds(c * CHUNK, CHUNK)]], rows_v.at[slot], gsem)

        def put(c, slot):
            pltpu.async_copy(
                rows_v.at[slot], out_hbm.at[pl.ds(base + c * CHUNK, CHUNK)], osem)

        def wait_gather():
            pltpu.make_async_copy(
                table_sp.at[pl.ds(0, CHUNK)], rows_v.at[0], gsem).wait()

        def wait_put():
            pltpu.make_async_copy(
                rows_v.at[0], out_hbm.at[pl.ds(base, CHUNK)], osem).wait()

        for b in range(NBUF):
            gather(b, b)
        wait_gather()
        put(0, 0)

        def body(c, _):
            wait_gather()
            put(c, lax.rem(c, NBUF))
            wait_put()
            gather(c + NBUF - 1, lax.rem(c - 1, NBUF))
            return _

        lax.fori_loop(1, chunks_per_w - NBUF + 1, body, None)
        for c in range(chunks_per_w - NBUF + 1, chunks_per_w):
            wait_gather()
            put(c, c % NBUF)
            wait_put()
        wait_put()

    return k(idx, table)


def kernel(op_indices, W):
    n = op_indices.shape[0]
    chunks_per_w = -(-n // (NW * CHUNK))
    idx = op_indices.astype(jnp.int32)
    return _sc_gather(idx, W, chunks_per_w=chunks_per_w)


# speedup vs baseline: 1.0089x; 1.0089x over previous
import functools

import jax
import jax.numpy as jnp
from jax import lax
from jax.experimental import pallas as pl
from jax.experimental.pallas import tpu as pltpu
from jax.experimental.pallas import tpu_sc as plsc

NC = 2
NS = 16
NW = NC * NS
CHUNK = 136
NBUF = 6


@functools.partial(jax.jit, static_argnames=("chunks_per_w",))
def _sc_gather(idx, table, *, chunks_per_w):
    n = idx.shape[0]
    d = table.shape[1]
    b_per_w = chunks_per_w * CHUNK
    assert chunks_per_w >= NBUF + 1
    assert n % 8 == 0 and b_per_w <= n
    stride = -(-(n - b_per_w) // (8 * (NW - 1))) * 8
    mesh = plsc.VectorSubcoreMesh(core_axis_name="c", subcore_axis_name="s")

    @functools.partial(
        pl.kernel,
        mesh=mesh,
        out_type=jax.ShapeDtypeStruct((n, d), jnp.float32),
        scratch_types=[
            pltpu.VMEM((b_per_w,), jnp.int32),
            pltpu.VMEM((NBUF, CHUNK, d), jnp.float32),
            pltpu.VMEM_SHARED(table.shape, jnp.float32),
            pltpu.SemaphoreType.DMA,
            pltpu.SemaphoreType.DMA,
        ],
    )
    def k(idx_hbm, table_hbm, out_hbm, idx_v, rows_v, table_sp, gsem, osem):
        sid = lax.axis_index("s")
        wid = sid * NC + lax.axis_index("c")
        base = pl.multiple_of(jnp.minimum(wid * stride, n - b_per_w), 8)

        @pl.when(sid == 0)
        def _stage():
            pltpu.sync_copy(table_hbm, table_sp)

        pltpu.sync_copy(idx_hbm.at[pl.ds(base, b_per_w)], idx_v)
        plsc.subcore_barrier()

        def gather(c, slot):
            pltpu.async_copy(
                table_sp.at[idx_v.at[pl.ds(c * CHUNK, CHUNK)]], rows_v.at[slot], gsem)

        def put(c, slot):
            pltpu.async_copy(
                rows_v.at[slot], out_hbm.at[pl.ds(base + c * CHUNK, CHUNK)], osem)

        def wait_gather():
            pltpu.make_async_copy(
                table_sp.at[pl.ds(0, CHUNK)], rows_v.at[0], gsem).wait()

        def wait_put():
            pltpu.make_async_copy(
                rows_v.at[0], out_hbm.at[pl.ds(base, CHUNK)], osem).wait()

        for b in range(NBUF):
            gather(b, b)
        wait_gather()
        put(0, 0)

        def body(c, _):
            wait_gather()
            put(c, lax.rem(c, NBUF))
            wait_put()
            gather(c + NBUF - 1, lax.rem(c - 1, NBUF))
            return _

        lax.fori_loop(1, chunks_per_w - NBUF + 1, body, None)
        for c in range(chunks_per_w - NBUF + 1, chunks_per_w):
            wait_gather()
            put(c, c % NBUF)
            wait_put()
        wait_put()

    return k(idx, table)


def kernel(op_indices, W):
    n = op_indices.shape[0]
    chunks_per_w = -(-n // (NW * CHUNK))
    idx = op_indices.astype(jnp.int32)
    return _sc_gather(idx, W, chunks_per_w=chunks_per_w)
